# Initial kernel scaffold; baseline (speedup 1.0000x reference)
import jax
import jax.numpy as jnp
from jax.experimental import pallas as pl


def kernel(loc_data, conf_data, priors, truths, labels):
    raise NotImplementedError("write your pallas kernel here")


# speedup vs baseline: 2.3439x; 2.3439x over previous
import functools

import jax
import jax.numpy as jnp
from jax.experimental import pallas as pl
from jax.experimental.pallas import tpu as pltpu

_VAR0 = 0.1
_VAR1 = 0.2
_K = 200
_SL1_WEIGHT = 0.75
_SL1_BETA = 0.11
_FOCAL_ALPHA = 0.5
_FOCAL_GAMMA = 2.0
_THRESHOLD = 0.5
_NUM_CLASSES = 21

_P = 8732
_PPAD = 8832
_NOBJ = 16
_CPAD = 24


def _iou_rows(bx1, by1, bx2, by2, area_b, tx1, ty1, tx2, ty2, area_t):
    ix = jnp.clip(jnp.minimum(bx2, tx2) - jnp.maximum(bx1, tx1), 0.0, None)
    iy = jnp.clip(jnp.minimum(by2, ty2) - jnp.maximum(by1, ty1), 0.0, None)
    inter = ix * iy
    union = area_b + area_t - inter
    return inter / union


def _loss_kernel(conf_ref, locp_ref, tr_ref, oh_ref, pos_ref, neg_ref):
    b = pl.program_id(0)

    lane = jax.lax.broadcasted_iota(jnp.int32, (1, _PPAD), 1)
    lane_valid = (lane < _P)

    conf = conf_ref[0]
    m = jnp.max(conf, axis=0, keepdims=True)
    e = jnp.exp(conf - m)
    cls_t = e / jnp.sum(e, axis=0, keepdims=True)

    lcx = locp_ref[0, 0:1, :]
    lcy = locp_ref[0, 1:2, :]
    lw = locp_ref[0, 2:3, :]
    lh = locp_ref[0, 3:4, :]
    pcx = locp_ref[0, 4:5, :]
    pcy = locp_ref[0, 5:6, :]
    pw = locp_ref[0, 6:7, :]
    ph = locp_ref[0, 7:8, :]

    tr = tr_ref[0]
    tx1 = tr[:, 0:1]
    ty1 = tr[:, 1:2]
    tx2 = tr[:, 2:3]
    ty2 = tr[:, 3:4]
    area_t = (tx2 - tx1) * (ty2 - ty1)

    dcx = pcx + lcx * (_VAR0) * pw
    dcy = pcy + lcy * (_VAR0) * ph
    dw = pw * jnp.exp(lw * _VAR1)
    dh = ph * jnp.exp(lh * _VAR1)
    ax1 = dcx - dw * 0.5
    ay1 = dcy - dh * 0.5
    ax2 = dcx + dw * 0.5
    ay2 = dcy + dh * 0.5
    area_a = dw * dh

    iou_a = _iou_rows(ax1, ay1, ax2, ay2, area_a, tx1, ty1, tx2, ty2, area_t)

    t2 = jnp.clip(jnp.max(iou_a, axis=1, keepdims=True), _THRESHOLD + 1e-12, None)
    iou_n = jnp.clip((iou_a - _THRESHOLD) / (t2 - _THRESHOLD), 0.0, 1.0)

    oh = oh_ref[0]
    bp = jnp.zeros((_CPAD, _PPAD), jnp.float32)
    for o in range(_NOBJ):
        bp = jnp.maximum(bp, oh[:, o:o + 1] * iou_n[o:o + 1, :])

    x = cls_t * (1.0 - bp)
    bce = -jnp.maximum(jnp.log1p(-x), -100.0)
    neg_partial = jnp.sum(jnp.where(lane_valid, x * x * bce, 0.0))

    px1 = pcx - pw * 0.5
    py1 = pcy - ph * 0.5
    px2 = pcx + pw * 0.5
    py2 = pcy + ph * 0.5
    area_p = pw * ph
    mqm = _iou_rows(px1, py1, px2, py2, area_p, tx1, ty1, tx2, ty2, area_t)
    mqm = jnp.where(lane_valid, mqm, -1.0)

    g_cx = ((tx1 + tx2) * 0.5 - pcx) / (_VAR0 * pw)
    g_cy = ((ty1 + ty2) * 0.5 - pcy) / (_VAR0 * ph)
    g_w = jnp.log((tx2 - tx1) / pw) * (1.0 / _VAR1)
    g_h = jnp.log((ty2 - ty1) / ph) * (1.0 / _VAR1)

    def sl1(v):
        a = jnp.abs(v)
        return jnp.where(a < _SL1_BETA, (0.5 / _SL1_BETA) * v * v, a - 0.5 * _SL1_BETA)

    reg = _SL1_WEIGHT * (sl1(g_cx - lcx) + sl1(g_cy - lcy)
                         + sl1(g_w - lw) + sl1(g_h - lh))
    box_p = jnp.exp(-reg)

    cls_sel = jax.lax.dot_general(oh, cls_t, (((0,), (0,)), ((), ())),
                                  preferred_element_type=jnp.float32)

    obj_iota = jax.lax.broadcasted_iota(jnp.int32, (_NOBJ, 1), 0)
    odd = (obj_iota & 1) == 1
    prs = []
    for r in range(_NOBJ):
        rv = jnp.where(odd, box_p[(r + 8) % _NOBJ:(r + 8) % _NOBJ + 1, :],
                       box_p[r:r + 1, :])
        prs.append(cls_sel * rv)

    n_outer = (_K + _NOBJ - 1) // _NOBJ

    def body(j, carry):
        vals, wsum, bags = carry
        for r in range(_NOBJ):
            m = jnp.max(vals, axis=1, keepdims=True)
            eq = vals == m
            idx = jnp.min(jnp.where(eq, lane, _PPAD), axis=1, keepdims=True)
            oneh = lane == idx
            lk = jnp.sum(jnp.where(oneh, prs[r], 0.0), axis=1, keepdims=True)
            vals = jnp.where(oneh, -2.0, vals)
            gate = ((j * _NOBJ + r) < _K).astype(jnp.float32)
            wk = gate / jnp.maximum(1.0 - lk, 1e-12)
            wsum = wsum + wk
            bags = bags + wk * lk
        return vals, wsum, bags

    zero = jnp.zeros((_NOBJ, 1), jnp.float32)
    _, wsum, bags = jax.lax.fori_loop(0, n_outer, body, (mqm, zero, zero))
    bag = bags / wsum
    pos_partial = jnp.sum(-jnp.maximum(jnp.log(bag), -100.0))

    @pl.when(b == 0)
    def _init():
        pos_ref[:, :] = jnp.zeros((1, 1), jnp.float32)
        neg_ref[:, :] = jnp.zeros((1, 1), jnp.float32)

    pos_ref[:, :] += pos_partial.reshape(1, 1)
    neg_ref[:, :] += neg_partial.reshape(1, 1)


@jax.jit
def kernel(loc_data, conf_data, priors, truths, labels):
    B = loc_data.shape[0]

    conf_t = jnp.transpose(conf_data, (0, 2, 1))
    conf_t = jnp.pad(conf_t, ((0, 0), (0, _CPAD - _NUM_CLASSES), (0, _PPAD - _P)),
                     constant_values=-1e30)

    loc_t = jnp.transpose(loc_data, (0, 2, 1))
    loc_t = jnp.pad(loc_t, ((0, 0), (0, 0), (0, _PPAD - _P)))
    pri_t = jnp.transpose(priors, (1, 0))
    pad_pri = jnp.tile(jnp.array([-100.0, -100.0, 1.0, 1.0], jnp.float32)[:, None],
                       (1, _PPAD - _P))
    pri_t = jnp.concatenate([pri_t, pad_pri], axis=1)
    locp = jnp.concatenate([loc_t, jnp.broadcast_to(pri_t, (B, 4, _PPAD))], axis=1)

    onehot = (labels[:, :, None] ==
              jnp.arange(_CPAD)[None, None, :]).astype(jnp.float32)
    oh_t = jnp.transpose(onehot, (0, 2, 1))

    grid_spec = pl.GridSpec(
        grid=(B,),
        in_specs=[
            pl.BlockSpec((1, _CPAD, _PPAD), lambda b: (b, 0, 0)),
            pl.BlockSpec((1, 8, _PPAD), lambda b: (b, 0, 0)),
            pl.BlockSpec((1, _NOBJ, 4), lambda b: (b, 0, 0)),
            pl.BlockSpec((1, _CPAD, _NOBJ), lambda b: (b, 0, 0)),
        ],
        out_specs=[
            pl.BlockSpec((1, 1), lambda b: (0, 0)),
            pl.BlockSpec((1, 1), lambda b: (0, 0)),
        ],
    )
    pos_sum, neg_sum = pl.pallas_call(
        _loss_kernel,
        grid_spec=grid_spec,
        out_shape=[jax.ShapeDtypeStruct((1, 1), jnp.float32),
                   jax.ShapeDtypeStruct((1, 1), jnp.float32)],
    )(conf_t, locp, truths, oh_t)

    denom = B * _NOBJ
    loss_p = pos_sum[0, 0] / denom * _FOCAL_ALPHA
    loss_n = neg_sum[0, 0] / (denom * _K) * (1.0 - _FOCAL_ALPHA)
    return (loss_p, loss_n)


# speedup vs baseline: 5.4796x; 2.3378x over previous
import functools

import jax
import jax.numpy as jnp
from jax import lax
from jax.experimental import pallas as pl
from jax.experimental.pallas import tpu as pltpu
from jax.experimental.pallas import tpu_sc as plsc

_VAR0 = 0.1
_VAR1 = 0.2
_K = 200
_SL1_WEIGHT = 0.75
_SL1_BETA = 0.11
_FOCAL_ALPHA = 0.5
_FOCAL_GAMMA = 2.0
_THRESHOLD = 0.5
_NUM_CLASSES = 21

_P = 8732
_PPAD = 8832
_NOBJ = 16
_CPAD = 24
_NC = 256


def _iou_rows(bx1, by1, bx2, by2, area_b, tx1, ty1, tx2, ty2, area_t):
    ix = jnp.clip(jnp.minimum(bx2, tx2) - jnp.maximum(bx1, tx1), 0.0, None)
    iy = jnp.clip(jnp.minimum(by2, ty2) - jnp.maximum(by1, ty1), 0.0, None)
    inter = ix * iy
    union = area_b + area_t - inter
    return inter / union


def _sl1(v):
    a = jnp.abs(v)
    return jnp.where(a < _SL1_BETA, (0.5 / _SL1_BETA) * v * v, a - 0.5 * _SL1_BETA)


def _stage1a_kernel(pri_ref, tr_ref, cand_ref):
    lane = jax.lax.broadcasted_iota(jnp.int32, (1, _PPAD), 1)
    lane_valid = (lane < _P)

    pcx = pri_ref[0:1, :]
    pcy = pri_ref[1:2, :]
    pw = pri_ref[2:3, :]
    ph = pri_ref[3:4, :]
    tr = tr_ref[0]
    tx1 = tr[:, 0:1]
    ty1 = tr[:, 1:2]
    tx2 = tr[:, 2:3]
    ty2 = tr[:, 3:4]
    area_t = (tx2 - tx1) * (ty2 - ty1)

    mqm = _iou_rows(pcx - pw * 0.5, pcy - ph * 0.5, pcx + pw * 0.5,
                    pcy + ph * 0.5, pw * ph, tx1, ty1, tx2, ty2, area_t)
    mqm = jnp.where(lane_valid, mqm, -1.0)

    lo = jnp.zeros((_NOBJ, 1), jnp.int32)
    hi = jnp.full((_NOBJ, 1), 0x3FC00000, jnp.int32)
    kf = jnp.float32(_NC)
    for _ in range(30):
        mid = (lo + hi) >> 1
        theta = jax.lax.bitcast_convert_type(mid, jnp.float32)
        c = jnp.sum((mqm >= theta).astype(jnp.float32), axis=1, keepdims=True)
        ge = c >= kf
        lo = jnp.where(ge, mid, lo)
        hi = jnp.where(ge, hi, mid)
    tstar = jax.lax.bitcast_convert_type(lo, jnp.float32)

    cnt_gt = jnp.sum((mqm > tstar).astype(jnp.float32), axis=1, keepdims=True)
    n_tie = kf - cnt_gt
    is_tie = (mqm == tstar)
    tie_f = is_tie.astype(jnp.float32)
    lo_j = jnp.full((_NOBJ, 1), -1, jnp.int32)
    hi_j = jnp.full((_NOBJ, 1), _PPAD - 1, jnp.int32)
    for _ in range(14):
        mid = (lo_j + hi_j) >> 1
        c = jnp.sum(jnp.where(lane <= mid, tie_f, 0.0), axis=1, keepdims=True)
        ge = c >= n_tie
        hi_j = jnp.where(ge, mid, hi_j)
        lo_j = jnp.where(ge, lo_j, mid)
    sel = (mqm > tstar) | (is_tie & (lane <= hi_j))
    cand_ref[0] = jnp.where(sel, mqm, -1.0)


def _stage1b_kernel(conf_ref, locp_ref, tr_ref, oh_ref, neg_ref):
    b = pl.program_id(0)

    lane = jax.lax.broadcasted_iota(jnp.int32, (1, _PPAD), 1)
    lane_valid = (lane < _P)

    conf = conf_ref[0]
    m = jnp.max(conf, axis=0, keepdims=True)
    e = jnp.exp(conf - m)
    cls_t = e / jnp.sum(e, axis=0, keepdims=True)

    lcx = locp_ref[0, 0:1, :]
    lcy = locp_ref[0, 1:2, :]
    lw = locp_ref[0, 2:3, :]
    lh = locp_ref[0, 3:4, :]
    pcx = locp_ref[0, 4:5, :]
    pcy = locp_ref[0, 5:6, :]
    pw = locp_ref[0, 6:7, :]
    ph = locp_ref[0, 7:8, :]

    tr = tr_ref[0]
    tx1 = tr[:, 0:1]
    ty1 = tr[:, 1:2]
    tx2 = tr[:, 2:3]
    ty2 = tr[:, 3:4]
    area_t = (tx2 - tx1) * (ty2 - ty1)

    dcx = pcx + lcx * _VAR0 * pw
    dcy = pcy + lcy * _VAR0 * ph
    dw = pw * jnp.exp(lw * _VAR1)
    dh = ph * jnp.exp(lh * _VAR1)
    iou_a = _iou_rows(dcx - dw * 0.5, dcy - dh * 0.5, dcx + dw * 0.5,
                      dcy + dh * 0.5, dw * dh, tx1, ty1, tx2, ty2, area_t)
    t2 = jnp.clip(jnp.max(iou_a, axis=1, keepdims=True), _THRESHOLD + 1e-12, None)
    iou_n = jnp.clip((iou_a - _THRESHOLD) / (t2 - _THRESHOLD), 0.0, 1.0)

    oh = oh_ref[0]
    bp = jnp.zeros((_CPAD, _PPAD), jnp.float32)
    for o in range(_NOBJ):
        bp = jnp.maximum(bp, oh[:, o:o + 1] * iou_n[o:o + 1, :])

    x = cls_t * (1.0 - bp)
    bce = -jnp.maximum(jnp.log1p(-x), -100.0)
    neg_partial = jnp.sum(jnp.where(lane_valid, x * x * bce, 0.0))

    @pl.when(b == 0)
    def _init():
        neg_ref[:, :] = jnp.zeros((1, 1), jnp.float32)

    neg_ref[:, :] += neg_partial.reshape(1, 1)


def _make_stage2(nrows):
    mesh = plsc.VectorSubcoreMesh(core_axis_name="c", subcore_axis_name="s")
    rows_per_worker = nrows // 32

    @functools.partial(
        pl.kernel,
        mesh=mesh,
        compiler_params=pltpu.CompilerParams(
            needs_layout_passes=False, use_tc_tiling_on_sc=False),
        out_type=[
            jax.ShapeDtypeStruct((nrows * _NC,), jnp.float32),
            jax.ShapeDtypeStruct((nrows * _NC,), jnp.int32),
            jax.ShapeDtypeStruct((nrows * _NC, 32), jnp.float32),
        ],
        scratch_types=[
            pltpu.VMEM((_PPAD,), jnp.float32),
            pltpu.VMEM((_NC,), jnp.float32),
            pltpu.VMEM((_NC,), jnp.int32),
            pltpu.VMEM((2, 128), jnp.int32),
            pltpu.VMEM((_NC, 32), jnp.float32),
            pltpu.SemaphoreType.DMA,
        ],
    )
    def k2(cand_hbm, packed_hbm, val_o, idx_o, raw_o,
           cand_v, val_v, idx_v, gidx_v, raw_v, sem):
        wid = lax.axis_index("s") * 2 + lax.axis_index("c")
        lane16 = jax.lax.broadcasted_iota(jnp.int32, (16,), 0)
        for t in range(rows_per_worker):
            row = wid * rows_per_worker + t
            b = row // _NOBJ
            pltpu.sync_copy(cand_hbm.at[pl.ds(row * _PPAD, _PPAD)], cand_v)

            def cbody(i, ptr):
                v = cand_v[pl.ds(i * 16, 16)]
                msk = v >= 0.0
                mi = msk.astype(jnp.int32)
                pc = lax.cumsum(mi, axis=0)
                pos = (ptr - 1) + pc
                plsc.store_scatter(val_v, [pos], v, mask=msk)
                gi = lane16 + i * 16
                plsc.store_scatter(idx_v, [pos], gi, mask=msk)
                return ptr + jnp.sum(mi)

            lax.fori_loop(0, _PPAD // 16, cbody, jnp.int32(0))

            for h in range(2):
                for q in range(8):
                    seg = idx_v[pl.ds(h * 128 + q * 16, 16)]
                    gidx_v[h, pl.ds(q * 16, 16)] = seg + b * _PPAD
            for h in range(2):
                pltpu.async_copy(packed_hbm.at[gidx_v.at[h]],
                                 raw_v.at[pl.ds(h * 128, 128)], sem).wait()

            pltpu.sync_copy(val_v, val_o.at[pl.ds(row * _NC, _NC)])
            pltpu.sync_copy(idx_v, idx_o.at[pl.ds(row * _NC, _NC)])
            pltpu.sync_copy(raw_v, raw_o.at[pl.ds(row * _NC, _NC)])

    return k2


def _stage3_kernel(val_ref, idx_ref, raw_ref, trt_ref, lab_ref, pos_ref):
    b = pl.program_id(0)
    n_all = _NOBJ * _NC

    val = val_ref[0]
    idxf = idx_ref[0].astype(jnp.float32)
    valt = jnp.transpose(val)
    idxt = jnp.transpose(idxf)

    ranks = []
    for o in range(_NOBJ):
        vrow = val[o:o + 1, :]
        irow = idxf[o:o + 1, :]
        vcol = valt[:, o:o + 1]
        icol = idxt[:, o:o + 1]
        before = (vrow > vcol) | ((vrow == vcol) & (irow < icol))
        ranks.append(jnp.sum(before.astype(jnp.float32), axis=1, keepdims=True))
    ranki = jnp.concatenate(ranks, axis=0).astype(jnp.int32)
    gate = (ranki < _K).astype(jnp.float32)
    par8 = (jax.lax.broadcasted_iota(jnp.int32, (n_all, 1), 0) >> 8) & 1
    tsel = (ranki + 8 * par8) & 15

    tT = trt_ref[0]
    lane16 = jax.lax.broadcasted_iota(jnp.int32, (1, _NOBJ), 1)
    lane32 = jax.lax.broadcasted_iota(jnp.int32, (1, 32), 1)
    colm = lane32 < _CPAD

    tm = (tsel == lane16).astype(jnp.float32)
    tx1 = jnp.sum(tm * tT[0:1, :], axis=1, keepdims=True)
    ty1 = jnp.sum(tm * tT[1:2, :], axis=1, keepdims=True)
    tx2 = jnp.sum(tm * tT[2:3, :], axis=1, keepdims=True)
    ty2 = jnp.sum(tm * tT[3:4, :], axis=1, keepdims=True)

    rawa = raw_ref[0]
    cm = jnp.max(jnp.where(colm, rawa, -1e30), axis=1, keepdims=True)
    ex = jnp.where(colm, jnp.exp(rawa - cm), 0.0)
    s = jnp.sum(ex, axis=1, keepdims=True)
    ohl = (lab_ref[0] == lane32).astype(jnp.float32)
    clsj = jnp.sum(ex * ohl, axis=1, keepdims=True) / s

    lcx = rawa[:, 24:25]
    lcy = rawa[:, 25:26]
    lw = rawa[:, 26:27]
    lh = rawa[:, 27:28]
    pcx = rawa[:, 28:29]
    pcy = rawa[:, 29:30]
    pw = rawa[:, 30:31]
    ph = rawa[:, 31:32]

    g_cx = ((tx1 + tx2) * 0.5 - pcx) / (_VAR0 * pw)
    g_cy = ((ty1 + ty2) * 0.5 - pcy) / (_VAR0 * ph)
    g_w = jnp.log((tx2 - tx1) / pw) * (1.0 / _VAR1)
    g_h = jnp.log((ty2 - ty1) / ph) * (1.0 / _VAR1)
    reg = _SL1_WEIGHT * (_sl1(g_cx - lcx) + _sl1(g_cy - lcy)
                         + _sl1(g_w - lw) + _sl1(g_h - lh))
    lgt = clsj * jnp.exp(-reg)
    w = gate / jnp.maximum(1.0 - lgt, 1e-12)
    wl = w * lgt

    pos_partial = jnp.zeros((), jnp.float32)
    for o in range(_NOBJ):
        bag = (jnp.sum(wl[o * _NC:(o + 1) * _NC, :])
               / jnp.sum(w[o * _NC:(o + 1) * _NC, :]))
        pos_partial += -jnp.maximum(jnp.log(bag), -100.0)

    @pl.when(b == 0)
    def _init():
        pos_ref[:, :] = jnp.zeros((1, 1), jnp.float32)

    pos_ref[:, :] += pos_partial.reshape(1, 1)


def _stage1a(pri8, truths, B):
    grid_spec = pl.GridSpec(
        grid=(B,),
        in_specs=[
            pl.BlockSpec((8, _PPAD), lambda b: (0, 0)),
            pl.BlockSpec((1, _NOBJ, 4), lambda b: (b, 0, 0)),
        ],
        out_specs=[pl.BlockSpec((1, _NOBJ, _PPAD), lambda b: (b, 0, 0))],
    )
    return pl.pallas_call(
        _stage1a_kernel,
        grid_spec=grid_spec,
        out_shape=[jax.ShapeDtypeStruct((B, _NOBJ, _PPAD), jnp.float32)],
    )(pri8, truths)[0]


def _stage1b(conf_t, locp, truths, oh_t, B):
    grid_spec = pl.GridSpec(
        grid=(B,),
        in_specs=[
            pl.BlockSpec((1, _CPAD, _PPAD), lambda b: (b, 0, 0)),
            pl.BlockSpec((1, 8, _PPAD), lambda b: (b, 0, 0)),
            pl.BlockSpec((1, _NOBJ, 4), lambda b: (b, 0, 0)),
            pl.BlockSpec((1, _CPAD, _NOBJ), lambda b: (b, 0, 0)),
        ],
        out_specs=[pl.BlockSpec((1, 1), lambda b: (0, 0))],
    )
    return pl.pallas_call(
        _stage1b_kernel,
        grid_spec=grid_spec,
        out_shape=[jax.ShapeDtypeStruct((1, 1), jnp.float32)],
    )(conf_t, locp, truths, oh_t)[0]


def _stage3(val, idx, raw, tr_t, ohl, B):
    grid_spec = pl.GridSpec(
        grid=(B,),
        in_specs=[
            pl.BlockSpec((1, _NOBJ, _NC), lambda b: (b, 0, 0)),
            pl.BlockSpec((1, _NOBJ, _NC), lambda b: (b, 0, 0)),
            pl.BlockSpec((1, _NOBJ * _NC, 32), lambda b: (b, 0, 0)),
            pl.BlockSpec((1, 4, _NOBJ), lambda b: (b, 0, 0)),
            pl.BlockSpec((1, _NOBJ * _NC, 1), lambda b: (b, 0, 0)),
        ],
        out_specs=[pl.BlockSpec((1, 1), lambda b: (0, 0))],
    )
    return pl.pallas_call(
        _stage3_kernel,
        grid_spec=grid_spec,
        out_shape=[jax.ShapeDtypeStruct((1, 1), jnp.float32)],
    )(val, idx, raw, tr_t, ohl)


@jax.jit
def kernel(loc_data, conf_data, priors, truths, labels):
    B = loc_data.shape[0]
    nrows = B * _NOBJ

    conf_t = jnp.transpose(conf_data, (0, 2, 1))
    conf_t = jnp.pad(conf_t, ((0, 0), (0, _CPAD - _NUM_CLASSES), (0, _PPAD - _P)),
                     constant_values=-1e30)
    loc_t = jnp.transpose(loc_data, (0, 2, 1))
    loc_t = jnp.pad(loc_t, ((0, 0), (0, 0), (0, _PPAD - _P)))
    pri_t = jnp.transpose(priors, (1, 0))
    pad_pri = jnp.tile(jnp.array([-100.0, -100.0, 1.0, 1.0], jnp.float32)[:, None],
                       (1, _PPAD - _P))
    pri_t = jnp.concatenate([pri_t, pad_pri], axis=1)
    locp = jnp.concatenate([loc_t, jnp.broadcast_to(pri_t, (B, 4, _PPAD))], axis=1)

    onehot = (labels[:, :, None] ==
              jnp.arange(_CPAD)[None, None, :]).astype(jnp.float32)
    oh_t = jnp.transpose(onehot, (0, 2, 1))

    conf_p = jnp.pad(conf_data, ((0, 0), (0, _PPAD - _P), (0, _CPAD - _NUM_CLASSES)),
                     constant_values=-1e30)
    loc_p = jnp.pad(loc_data, ((0, 0), (0, _PPAD - _P), (0, 0)))
    pri_p = jnp.pad(priors, ((0, _PPAD - _P), (0, 0)), constant_values=1.0)
    packed = jnp.concatenate(
        [conf_p, loc_p, jnp.broadcast_to(pri_p[None], (B, _PPAD, 4))],
        axis=-1).reshape(B * _PPAD, 32)

    pri8 = jnp.concatenate([pri_t, jnp.zeros((4, _PPAD), jnp.float32)], axis=0)
    cand = _stage1a(pri8, truths, B)
    val, idx, raw = _make_stage2(nrows)(cand.reshape(nrows * _PPAD), packed)
    neg_sum = _stage1b(conf_t, locp, truths, oh_t, B)

    tr_t = jnp.transpose(truths, (0, 2, 1))
    labcol = jnp.repeat(labels.astype(jnp.int32), _NC, axis=1)[:, :, None]

    pos_sum = _stage3(val.reshape(B, _NOBJ, _NC), idx.reshape(B, _NOBJ, _NC),
                      raw.reshape(B, _NOBJ * _NC, 32), tr_t, labcol, B)[0]

    denom = B * _NOBJ
    loss_p = pos_sum[0, 0] / denom * _FOCAL_ALPHA
    loss_n = neg_sum[0, 0] / (denom * _K) * (1.0 - _FOCAL_ALPHA)
    return (loss_p, loss_n)
